# Initial kernel scaffold; baseline (speedup 1.0000x reference)
import jax
import jax.numpy as jnp
from jax.experimental import pallas as pl


def kernel(x, dyvis_mask, Wq, bq, Wk, bk, Wv, bv, Wd, Wo):
    raise NotImplementedError("write your pallas kernel here")


# speedup vs baseline: 1.4253x; 1.4253x over previous
import functools

import jax
import jax.numpy as jnp
from jax.experimental import pallas as pl
from jax.experimental.pallas import tpu as pltpu

_PAD = 128


def _col_masks(W, HW):
    f32 = jnp.float32
    col_r = jax.lax.broadcasted_iota(jnp.int32, (1, HW), 1) % W
    mLr = (col_r >= 1).astype(f32)
    mRr = (col_r <= W - 2).astype(f32)
    return mLr, mRr


def _attn_body(x_ref, m_ref, wqkv_ref, bqkv_ref, wd_ref, ao_ref,
               slab_ref, qkv_ref, dmr_ref, dmc_ref,
               *, C, W, HW, CQK, NQKV, CV, BI):
    f32 = jnp.float32
    SLAB = HW + 2 * _PAD

    slab_ref[:, 0:_PAD] = jnp.zeros((C, _PAD), f32)
    slab_ref[:, _PAD + HW:SLAB] = jnp.zeros((C, _PAD), f32)

    mLr, mRr = _col_masks(W, HW)
    col_c = jax.lax.broadcasted_iota(jnp.int32, (HW, 1), 0) % W
    mLc = (col_c >= 1).astype(f32)
    mRc = (col_c <= W - 2).astype(f32)

    def slab(o):
        return slab_ref[:, _PAD + o:_PAD + o + HW]

    slab_ref[:, _PAD:_PAD + HW] = m_ref[0]
    accr = jnp.zeros((8, HW), f32)
    accc = jnp.zeros((HW, 8), f32)
    for kx in range(3):
        dx = kx - 1
        pr = jnp.zeros((8, HW), f32)
        pc = jnp.zeros((HW, 8), f32)
        for ky in range(3):
            s = ky * 3 + kx
            o = (ky - 1) * W + dx
            xs = slab(o)
            pr = pr + jnp.dot(wd_ref[s], xs, preferred_element_type=f32)
            pc = pc + jax.lax.dot_general(
                xs, wd_ref[s], (((0,), (1,)), ((), ())),
                preferred_element_type=f32)
        if dx == -1:
            pr, pc = pr * mLr, pc * mLc
        elif dx == 1:
            pr, pc = pr * mRr, pc * mRc
        accr = accr + pr
        accc = accc + pc
    dmr_ref[...] = jnp.clip(accr, 0.0, 1.0)
    dmc_ref[...] = jnp.clip(accc, 0.0, 1.0)

    slab_ref[:, _PAD:_PAD + HW] = x_ref[0]
    acc = jnp.zeros((NQKV, HW), f32)
    for kx in range(3):
        dx = kx - 1
        part = jnp.zeros((NQKV, HW), f32)
        for ky in range(3):
            s = ky * 3 + kx
            o = (ky - 1) * W + dx
            part = part + jnp.dot(wqkv_ref[s], slab(o),
                                  preferred_element_type=f32)
        if dx == -1:
            part = part * mLr
        elif dx == 1:
            part = part * mRr
        acc = acc + part
    qkv_ref[...] = acc + bqkv_ref[...]

    mm = dmr_ref[0:1, :]
    for i0 in range(0, HW, BI):
        qs = qkv_ref[0:CQK, i0:i0 + BI]
        k = qkv_ref[CQK:2 * CQK, :]
        e = jax.lax.dot_general(qs, k, (((0,), (0,)), ((), ())),
                                preferred_element_type=f32)
        im = 1.0 - dmc_ref[i0:i0 + BI, 0:1]
        e = e * mm * im
        emax = jnp.max(e, axis=1, keepdims=True)
        p = jnp.exp(e - emax)
        l = jnp.sum(p, axis=1, keepdims=True)
        p = p * (1.0 / l)
        v = qkv_ref[2 * CQK:NQKV, :]
        ob = jax.lax.dot_general(v, p, (((1,), (1,)), ((), ())),
                                 preferred_element_type=f32)
        ao_ref[0, :, i0:i0 + BI] = ob


def _out_body(x_ref, a_ref, wo_ref, o_ref, cat_ref, *, C, W, HW, CO, C2):
    f32 = jnp.float32
    SLAB = HW + 2 * _PAD

    cat_ref[:, 0:_PAD] = jnp.zeros((C2, _PAD), f32)
    cat_ref[:, _PAD + HW:SLAB] = jnp.zeros((C2, _PAD), f32)
    cat_ref[0:C, _PAD:_PAD + HW] = x_ref[0]
    cat_ref[C:C2, _PAD:_PAD + HW] = a_ref[0]

    mLr, mRr = _col_masks(W, HW)

    acc = jnp.zeros((CO, HW), f32)
    for kx in range(3):
        dx = kx - 1
        part = jnp.zeros((CO, HW), f32)
        for ky in range(3):
            s = ky * 3 + kx
            o = (ky - 1) * W + dx
            part = part + jnp.dot(wo_ref[s], cat_ref[:, _PAD + o:_PAD + o + HW],
                                  preferred_element_type=f32)
        if dx == -1:
            part = part * mLr
        elif dx == 1:
            part = part * mRr
        acc = acc + part
    o_ref[0] = acc


def kernel(x, dyvis_mask, Wq, bq, Wk, bk, Wv, bv, Wd, Wo):
    B, C, H, W = x.shape
    HW = H * W
    CQK = Wq.shape[0]
    CV = Wv.shape[0]
    CO, C2 = Wo.shape[0], Wo.shape[1]
    NQKV = 2 * CQK + CV
    BI = 256 if HW % 256 == 0 else HW
    SLAB = HW + 2 * _PAD

    x3 = x.reshape(B, C, HW)
    m3 = dyvis_mask.reshape(B, C, HW)
    wqkv = jnp.concatenate([Wq, Wk, Wv], axis=0).transpose(2, 3, 0, 1)
    wqkv = wqkv.reshape(9, NQKV, C)
    bqkv = jnp.concatenate([bq, bk, bv]).reshape(NQKV, 1)
    wd9 = jnp.pad(Wd, ((0, 7), (0, 0), (0, 0), (0, 0)))
    wd9 = wd9.transpose(2, 3, 0, 1).reshape(9, 8, C)
    wo9 = Wo.transpose(2, 3, 0, 1).reshape(9, CO, C2)

    attn_body = functools.partial(_attn_body, C=C, W=W, HW=HW, CQK=CQK,
                                  NQKV=NQKV, CV=CV, BI=BI)
    attn = pl.pallas_call(
        attn_body,
        grid=(B,),
        in_specs=[
            pl.BlockSpec((1, C, HW), lambda b: (b, 0, 0)),
            pl.BlockSpec((1, C, HW), lambda b: (b, 0, 0)),
            pl.BlockSpec((9, NQKV, C), lambda b: (0, 0, 0)),
            pl.BlockSpec((NQKV, 1), lambda b: (0, 0)),
            pl.BlockSpec((9, 8, C), lambda b: (0, 0, 0)),
        ],
        out_specs=pl.BlockSpec((1, CV, HW), lambda b: (b, 0, 0)),
        out_shape=jax.ShapeDtypeStruct((B, CV, HW), jnp.float32),
        scratch_shapes=[
            pltpu.VMEM((C, SLAB), jnp.float32),
            pltpu.VMEM((NQKV, HW), jnp.float32),
            pltpu.VMEM((8, HW), jnp.float32),
            pltpu.VMEM((HW, 8), jnp.float32),
        ],
        compiler_params=pltpu.CompilerParams(
            dimension_semantics=("parallel",),
            vmem_limit_bytes=58 * 1024 * 1024,
        ),
    )(x3, m3, wqkv, bqkv, wd9)

    out_body = functools.partial(_out_body, C=C, W=W, HW=HW, CO=CO, C2=C2)
    out = pl.pallas_call(
        out_body,
        grid=(B,),
        in_specs=[
            pl.BlockSpec((1, C, HW), lambda b: (b, 0, 0)),
            pl.BlockSpec((1, CV, HW), lambda b: (b, 0, 0)),
            pl.BlockSpec((9, CO, C2), lambda b: (0, 0, 0)),
        ],
        out_specs=pl.BlockSpec((1, CO, HW), lambda b: (b, 0, 0)),
        out_shape=jax.ShapeDtypeStruct((B, CO, HW), jnp.float32),
        scratch_shapes=[
            pltpu.VMEM((C2, SLAB), jnp.float32),
        ],
        compiler_params=pltpu.CompilerParams(
            dimension_semantics=("parallel",),
            vmem_limit_bytes=58 * 1024 * 1024,
        ),
    )(x3, attn, wo9)
    return out.reshape(B, CO, H, W)


# speedup vs baseline: 1.5068x; 1.0572x over previous
import functools

import jax
import jax.numpy as jnp
from jax.experimental import pallas as pl
from jax.experimental.pallas import tpu as pltpu

_PAD = 128


def _body(x_ref, m_ref, wqkv_ref, bqkv_ref, wd_ref, wo_ref, o_ref,
          cat_ref, qkv_ref, dmr_ref, dmc_ref,
          *, C, W, HW, CQK, NQKV, CV, CO, C2, BI):
    f32 = jnp.float32
    bf16 = jnp.bfloat16
    SLAB = HW + 2 * _PAD

    cat_ref[:, 0:_PAD] = jnp.zeros((C2, _PAD), bf16)
    cat_ref[:, _PAD + HW:SLAB] = jnp.zeros((C2, _PAD), bf16)

    col_r = jax.lax.broadcasted_iota(jnp.int32, (1, HW), 1) % W
    mLr = (col_r >= 1).astype(f32)
    mRr = (col_r <= W - 2).astype(f32)
    col_c = jax.lax.broadcasted_iota(jnp.int32, (HW, 1), 0) % W
    mLc = (col_c >= 1).astype(f32)
    mRc = (col_c <= W - 2).astype(f32)

    def slab(o):
        return cat_ref[0:C, _PAD + o:_PAD + o + HW]

    cat_ref[0:C, _PAD:_PAD + HW] = m_ref[0]
    accr = jnp.zeros((8, HW), f32)
    accc = jnp.zeros((HW, 8), f32)
    for kx in range(3):
        dx = kx - 1
        pr = jnp.zeros((8, HW), f32)
        pc = jnp.zeros((HW, 8), f32)
        for ky in range(3):
            s = ky * 3 + kx
            o = (ky - 1) * W + dx
            xs = slab(o)
            pr = pr + jnp.dot(wd_ref[s], xs, preferred_element_type=f32)
            pc = pc + jax.lax.dot_general(
                xs, wd_ref[s], (((0,), (1,)), ((), ())),
                preferred_element_type=f32)
        if dx == -1:
            pr, pc = pr * mLr, pc * mLc
        elif dx == 1:
            pr, pc = pr * mRr, pc * mRc
        accr = accr + pr
        accc = accc + pc
    dmr_ref[...] = jnp.clip(accr, 0.0, 1.0)
    dmc_ref[...] = jnp.clip(accc, 0.0, 1.0)

    cat_ref[0:C, _PAD:_PAD + HW] = x_ref[0]
    acc = jnp.zeros((NQKV, HW), f32)
    for kx in range(3):
        dx = kx - 1
        part = jnp.zeros((NQKV, HW), f32)
        for ky in range(3):
            s = ky * 3 + kx
            o = (ky - 1) * W + dx
            part = part + jnp.dot(wqkv_ref[s], slab(o),
                                  preferred_element_type=f32)
        if dx == -1:
            part = part * mLr
        elif dx == 1:
            part = part * mRr
        acc = acc + part
    qkv_ref[...] = (acc + bqkv_ref[...]).astype(bf16)

    mm = dmr_ref[0:1, :]
    for i0 in range(0, HW, BI):
        qs = qkv_ref[0:CQK, i0:i0 + BI]
        k = qkv_ref[CQK:2 * CQK, :]
        e = jax.lax.dot_general(qs, k, (((0,), (0,)), ((), ())),
                                preferred_element_type=f32)
        im = 1.0 - dmc_ref[i0:i0 + BI, 0:1]
        e = e * mm * im
        emax = jnp.max(e, axis=1, keepdims=True)
        p = jnp.exp(e - emax)
        l = jnp.sum(p, axis=1, keepdims=True)
        pb = (p * (1.0 / l)).astype(bf16)
        v = qkv_ref[2 * CQK:NQKV, :]
        ob = jax.lax.dot_general(v, pb, (((1,), (1,)), ((), ())),
                                 preferred_element_type=f32)
        cat_ref[C:C + CV, _PAD + i0:_PAD + i0 + BI] = ob.astype(bf16)

    acco = jnp.zeros((CO, HW), f32)
    for kx in range(3):
        dx = kx - 1
        part = jnp.zeros((CO, HW), f32)
        for ky in range(3):
            s = ky * 3 + kx
            o = (ky - 1) * W + dx
            part = part + jnp.dot(wo_ref[s], cat_ref[:, _PAD + o:_PAD + o + HW],
                                  preferred_element_type=f32)
        if dx == -1:
            part = part * mLr
        elif dx == 1:
            part = part * mRr
        acco = acco + part
    o_ref[0] = acco


def kernel(x, dyvis_mask, Wq, bq, Wk, bk, Wv, bv, Wd, Wo):
    B, C, H, W = x.shape
    HW = H * W
    CQK = Wq.shape[0]
    CV = Wv.shape[0]
    CO, C2 = Wo.shape[0], Wo.shape[1]
    NQKV = 2 * CQK + CV
    BI = 256 if HW % 256 == 0 else HW
    SLAB = HW + 2 * _PAD
    bf16 = jnp.bfloat16

    x3 = x.reshape(B, C, HW).astype(bf16)
    m3 = dyvis_mask.reshape(B, C, HW).astype(bf16)
    wqkv = jnp.concatenate([Wq, Wk, Wv], axis=0).transpose(2, 3, 0, 1)
    wqkv = wqkv.reshape(9, NQKV, C).astype(bf16)
    bqkv = jnp.concatenate([bq, bk, bv]).reshape(NQKV, 1)
    wd9 = jnp.pad(Wd, ((0, 7), (0, 0), (0, 0), (0, 0)))
    wd9 = wd9.transpose(2, 3, 0, 1).reshape(9, 8, C).astype(bf16)
    wo9 = Wo.transpose(2, 3, 0, 1).reshape(9, CO, C2).astype(bf16)

    body = functools.partial(_body, C=C, W=W, HW=HW, CQK=CQK, NQKV=NQKV,
                             CV=CV, CO=CO, C2=C2, BI=BI)
    out = pl.pallas_call(
        body,
        grid=(B,),
        in_specs=[
            pl.BlockSpec((1, C, HW), lambda b: (b, 0, 0)),
            pl.BlockSpec((1, C, HW), lambda b: (b, 0, 0)),
            pl.BlockSpec((9, NQKV, C), lambda b: (0, 0, 0)),
            pl.BlockSpec((NQKV, 1), lambda b: (0, 0)),
            pl.BlockSpec((9, 8, C), lambda b: (0, 0, 0)),
            pl.BlockSpec((9, CO, C2), lambda b: (0, 0, 0)),
        ],
        out_specs=pl.BlockSpec((1, CO, HW), lambda b: (b, 0, 0)),
        out_shape=jax.ShapeDtypeStruct((B, CO, HW), jnp.float32),
        scratch_shapes=[
            pltpu.VMEM((C2, SLAB), bf16),
            pltpu.VMEM((NQKV, HW), bf16),
            pltpu.VMEM((8, HW), jnp.float32),
            pltpu.VMEM((HW, 8), jnp.float32),
        ],
        compiler_params=pltpu.CompilerParams(
            dimension_semantics=("parallel",),
            vmem_limit_bytes=58 * 1024 * 1024,
        ),
    )(x3, m3, wqkv, bqkv, wd9, wo9)
    return out.reshape(B, CO, H, W)
